# Initial kernel scaffold; baseline (speedup 1.0000x reference)
import jax
import jax.numpy as jnp
from jax.experimental import pallas as pl


def kernel(x, w1, b1, w2, b2):
    raise NotImplementedError("write your pallas kernel here")


# speedup vs baseline: 1.2564x; 1.2564x over previous
import functools

import jax
import jax.numpy as jnp
from jax.experimental import pallas as pl
from jax.experimental.pallas import tpu as pltpu


def _se_fused_kernel(x_ref, w1_ref, b1_ref, w2_ref, b2_ref, o_ref, *, inv_hw):
    pooled = jnp.sum(x_ref[...].astype(jnp.float32), axis=-1) * inv_hw

    h = jnp.dot(pooled, w1_ref[...], preferred_element_type=jnp.float32)
    h = jnp.maximum(h + b1_ref[...], 0.0)
    z = jnp.dot(h, w2_ref[...], preferred_element_type=jnp.float32)
    z = z + b2_ref[...]
    s = jax.nn.sigmoid(z)

    o_ref[...] = (x_ref[...] * s[:, :, None]).astype(o_ref.dtype)


def kernel(x, w1, b1, w2, b2):
    N, C, H, W = x.shape
    HW = H * W
    Cr = w1.shape[1]
    itemsize = jnp.dtype(x.dtype).itemsize

    x_flat = x.reshape(N, C, HW)

    cost = pl.CostEstimate(
        flops=int(2 * N * C * HW + 4 * N * C * Cr),
        transcendentals=int(N * C),
        bytes_accessed=int(2 * N * C * HW * itemsize
                           + (C * Cr + Cr + Cr * C + C) * 4),
    )

    out_flat = pl.pallas_call(
        functools.partial(_se_fused_kernel, inv_hw=1.0 / HW),
        out_shape=jax.ShapeDtypeStruct((N, C, HW), x.dtype),
        grid=(N,),
        in_specs=[
            pl.BlockSpec((1, C, HW), lambda n: (n, 0, 0)),
            pl.BlockSpec((C, Cr), lambda n: (0, 0)),
            pl.BlockSpec((1, Cr), lambda n: (0, 0)),
            pl.BlockSpec((Cr, C), lambda n: (0, 0)),
            pl.BlockSpec((1, C), lambda n: (0, 0)),
        ],
        out_specs=pl.BlockSpec((1, C, HW), lambda n: (n, 0, 0)),
        compiler_params=pltpu.CompilerParams(
            dimension_semantics=("parallel",),
            vmem_limit_bytes=48 * 1024 * 1024),
        cost_estimate=cost,
    )(x_flat, w1, b1, w2, b2)

    return out_flat.reshape(N, C, H, W)
